# baseline (speedup 1.0000x reference)
import functools

import jax
import jax.numpy as jnp
from jax import lax
from jax.experimental import pallas as pl
from jax.experimental.pallas import tpu as pltpu
from jax.experimental.pallas import tpu_sc as plsc

N_FIELDS = 26
VOCAB = 100000
DIM = 32
EPS = 1e-5
OUT_D = N_FIELDS * DIM
PAD_D = 896

L = 16
NC = 2
NS = 16
NW = NC * NS

C = 64
CE = C * N_FIELDS
NIDX = CE // 128
NVEC = CE // L
OFF_P = 208


def _rsqrt_vec(x):
    i = plsc.bitcast(x, jnp.int32)
    i = jnp.int32(0x5F3759DF) - lax.shift_right_logical(i, 1)
    y = plsc.bitcast(i, jnp.float32)
    for _ in range(3):
        y = y * (1.5 - 0.5 * x * y * y)
    return y


def _make_sc_call(B):
    assert B % (NW * C) == 0
    chunks = B // (NW * C)
    mesh = plsc.VectorSubcoreMesh(core_axis_name="c", subcore_axis_name="s")

    @functools.partial(
        pl.kernel,
        mesh=mesh,
        compiler_params=pltpu.CompilerParams(
            needs_layout_passes=False, use_tc_tiling_on_sc=False),
        out_type=jax.ShapeDtypeStruct((B, PAD_D), jnp.float32),
        scratch_types=[
            pltpu.VMEM((CE,), jnp.int32),
            pltpu.VMEM((NIDX, 128), jnp.int32),
            pltpu.VMEM((CE, DIM), jnp.float32),
            pltpu.VMEM((C, PAD_D), jnp.float32),
            pltpu.VMEM((OFF_P,), jnp.int32),
            pltpu.VMEM((OUT_D,), jnp.float32),
            pltpu.VMEM((OUT_D,), jnp.float32),
            pltpu.SemaphoreType.DMA,
        ],
    )
    def sc_call(tab, catf, off, gamma, beta, out,
                cat_v, idx_v, rows_v, out_v, off_v, g_v, b_v, sem):
        wid = lax.axis_index("s") * NC + lax.axis_index("c")
        pltpu.sync_copy(off, off_v)
        pltpu.sync_copy(gamma, g_v)
        pltpu.sync_copy(beta, b_v)

        def chunk_body(c, carry):
            row0 = (wid * chunks + c) * C
            e0 = row0 * N_FIELDS
            pltpu.sync_copy(catf.at[pl.ds(e0, CE)], cat_v)
            for i in range(NVEC):
                v = cat_v[pl.ds(i * L, L)]
                v = jnp.minimum(jnp.maximum(v, 0), VOCAB - 1)
                v = v + off_v[pl.ds((i % 13) * L, L)]
                idx_v[i // 8, pl.ds((i % 8) * L, L)] = v
            copies = [
                pltpu.async_copy(tab.at[idx_v.at[j]],
                                 rows_v.at[pl.ds(j * 128, 128)], sem)
                for j in range(NIDX)
            ]
            for cp in copies:
                cp.wait()

            def ln_body(r, carry2):
                base = r * N_FIELDS
                acc = jnp.zeros((L,), jnp.float32)
                acc2 = jnp.zeros((L,), jnp.float32)
                for k in range(N_FIELDS):
                    for h in range(2):
                        v = rows_v[base + k, pl.ds(h * L, L)]
                        acc = acc + v
                        acc2 = acc2 + v * v
                s = jnp.sum(acc)
                s2 = jnp.sum(acc2)
                mean = s * (1.0 / OUT_D)
                var = s2 * (1.0 / OUT_D) - mean * mean
                mean_v = jnp.full((L,), mean, jnp.float32)
                rstd_v = _rsqrt_vec(jnp.full((L,), var + EPS, jnp.float32))
                for k in range(N_FIELDS):
                    for h in range(2):
                        v = rows_v[base + k, pl.ds(h * L, L)]
                        gs = g_v[pl.ds(k * DIM + h * L, L)]
                        bs = b_v[pl.ds(k * DIM + h * L, L)]
                        out_v[r, pl.ds(k * DIM + h * L, L)] = (
                            (v - mean_v) * rstd_v * gs + bs)
                return carry2

            lax.fori_loop(0, C, ln_body, 0)
            pltpu.sync_copy(out_v, out.at[pl.ds(row0, C)])
            return carry

        lax.fori_loop(0, chunks, chunk_body, 0)

    return sc_call


def kernel(cat, tables, gamma, beta):
    B = cat.shape[0]
    tab, catf = jax.lax.optimization_barrier(
        (tables.reshape(N_FIELDS * VOCAB, DIM), cat.reshape(B * N_FIELDS)))
    off = (jnp.arange(OFF_P, dtype=jnp.int32) % N_FIELDS) * VOCAB
    out = _make_sc_call(B)(tab, catf, off, gamma, beta)
    return out[:, :OUT_D]


# speedup vs baseline: 3.4740x; 3.4740x over previous
import functools

import jax
import jax.numpy as jnp
from jax import lax
from jax.experimental import pallas as pl
from jax.experimental.pallas import tpu as pltpu
from jax.experimental.pallas import tpu_sc as plsc

N_FIELDS = 26
VOCAB = 100000
DIM = 32
EPS = 1e-5
OUT_D = N_FIELDS * DIM

L = 16
NC = 2
NS = 16
NW = NC * NS


def _make_sc_gather(B):
    assert DIM == NW
    HB = B // 2
    mesh = plsc.VectorSubcoreMesh(core_axis_name="c", subcore_axis_name="s")

    @functools.partial(
        pl.kernel,
        mesh=mesh,
        compiler_params=pltpu.CompilerParams(
            needs_layout_passes=False, use_tc_tiling_on_sc=True),
        out_type=jax.ShapeDtypeStruct((OUT_D, B), jnp.float32),
        scratch_types=[
            pltpu.VMEM((VOCAB,), jnp.float32),
            pltpu.VMEM((HB,), jnp.int32),
            pltpu.VMEM((B,), jnp.float32),
            pltpu.SemaphoreType.DMA,
        ],
    )
    def sc_gather(tabT, catT, out, plane_v, idx_v, res_v, sem):
        d = lax.axis_index("s") * NC + lax.axis_index("c")

        def fbody(f, carry):
            cp = pltpu.async_copy(tabT.at[f, d], plane_v, sem)
            pltpu.sync_copy(catT.at[f, pl.ds(0, HB)], idx_v)
            cp.wait()
            for h in range(2):
                if h == 1:
                    pltpu.sync_copy(catT.at[f, pl.ds(HB, HB)], idx_v)

                def gbody(i, c2, h=h):
                    iv = idx_v[pl.ds(i * L, L)]
                    iv = jnp.minimum(jnp.maximum(iv, 0), VOCAB - 1)
                    res_v[pl.ds(h * HB + i * L, L)] = plsc.load_gather(
                        plane_v, [iv])
                    return c2

                lax.fori_loop(0, HB // L, gbody, 0)
            pltpu.sync_copy(res_v, out.at[f * DIM + d])
            return carry

        lax.fori_loop(0, N_FIELDS, fbody, 0)

    return sc_gather


def _tc_layernorm(gath, gamma, beta):
    D, B = gath.shape
    BL = 512

    def ln_body(x_ref, g_ref, b_ref, o_ref):
        x = x_ref[...]
        mean = jnp.mean(x, axis=0, keepdims=True)
        xc = x - mean
        var = jnp.mean(xc * xc, axis=0, keepdims=True)
        r = lax.rsqrt(var + EPS)
        o_ref[...] = xc * r * g_ref[...] + b_ref[...]

    return pl.pallas_call(
        ln_body,
        grid=(B // BL,),
        in_specs=[
            pl.BlockSpec((D, BL), lambda i: (0, i)),
            pl.BlockSpec((D, 1), lambda i: (0, 0)),
            pl.BlockSpec((D, 1), lambda i: (0, 0)),
        ],
        out_specs=pl.BlockSpec((D, BL), lambda i: (0, i)),
        out_shape=jax.ShapeDtypeStruct((D, B), jnp.float32),
    )(gath, gamma.reshape(D, 1), beta.reshape(D, 1))


def kernel(cat, tables, gamma, beta):
    B = cat.shape[0]
    catT = cat.T
    tabT = tables.transpose(0, 2, 1)
    gath = _make_sc_gather(B)(tabT, catT)
    outT = _tc_layernorm(gath, gamma, beta)
    return outT.T
